# baseline (speedup 1.0000x reference)
import functools

import jax
import jax.numpy as jnp
from jax import lax
from jax.experimental import pallas as pl
from jax.experimental.pallas import tpu as pltpu
from jax.experimental.pallas import tpu_sc as plsc

NC = 2
NS = 16
L = 16
NW = NC * NS

B = 16384
D = 64
B_PER_W = B // NW
GROUPS = B_PER_W // L


def _body(i_hbm, j_hbm, h_hbm, c_hbm, out_hbm,
          idx_i, idx_j, h_rows, c_rows, out_v, sem):
    wid = lax.axis_index("s") * NC + lax.axis_index("c")
    base = wid * B_PER_W

    pltpu.sync_copy(i_hbm.at[pl.ds(base, B_PER_W)], idx_i)
    pltpu.sync_copy(j_hbm.at[pl.ds(base, B_PER_W)], idx_j)

    cp_h = pltpu.make_async_copy(h_hbm.at[idx_i], h_rows, sem)
    cp_c = pltpu.make_async_copy(c_hbm.at[idx_j], c_rows, sem)
    cp_h.start()
    cp_c.start()
    cp_h.wait()
    cp_c.wait()

    lane = lax.iota(jnp.int32, L)

    def group(g, _):
        rows = g * L + lane
        acc = jnp.zeros((L,), jnp.float32)
        dvec = jnp.zeros((L,), jnp.int32)
        for _step in range(D):
            hv = plsc.load_gather(h_rows, [rows, dvec])
            cv = plsc.load_gather(c_rows, [rows, dvec])
            acc = acc + hv * cv
            dvec = dvec + 1
        sig = 1.0 / (1.0 + jnp.exp(-acc))
        out_v[pl.ds(g * L, L)] = sig
        return ()

    lax.fori_loop(0, GROUPS, group, (), unroll=False)

    pltpu.sync_copy(out_v, out_hbm.at[pl.ds(base, B_PER_W)])


@functools.partial(jax.jit, static_argnames=())
def kernel(i, j, H, C):
    mesh = plsc.VectorSubcoreMesh(
        core_axis_name="c", subcore_axis_name="s",
        num_cores=NC, num_subcores=NS)
    run = pl.kernel(
        _body,
        out_type=jax.ShapeDtypeStruct((B,), jnp.float32),
        mesh=mesh,
        scratch_types=[
            pltpu.VMEM((B_PER_W,), jnp.int32),
            pltpu.VMEM((B_PER_W,), jnp.int32),
            pltpu.VMEM((B_PER_W, D), jnp.float32),
            pltpu.VMEM((B_PER_W, D), jnp.float32),
            pltpu.VMEM((B_PER_W,), jnp.float32),
            pltpu.SemaphoreType.DMA,
        ],
        compiler_params=pltpu.CompilerParams(
            needs_layout_passes=False, use_tc_tiling_on_sc=False),
    )
    return run(i.astype(jnp.int32), j.astype(jnp.int32), H, C)


# speedup vs baseline: 1.5542x; 1.5542x over previous
import jax
import jax.numpy as jnp
from jax import lax
from jax.experimental import pallas as pl
from jax.experimental.pallas import tpu as pltpu
from jax.experimental.pallas import tpu_sc as plsc

NC = 2
NS = 16
L = 16
NW = NC * NS

B = 16384
D = 64
B_PER_W = B // NW
CH = 128
NCHUNK = B_PER_W // CH
GPC = CH // L


def _body(i_hbm, j_hbm, h_hbm, c_hbm, out_hbm,
          idx_i, idx_j, hb, cb, out_v, sem):
    wid = lax.axis_index("s") * NC + lax.axis_index("c")
    base = wid * B_PER_W

    pltpu.sync_copy(i_hbm.at[pl.ds(base, B_PER_W)], idx_i)
    pltpu.sync_copy(j_hbm.at[pl.ds(base, B_PER_W)], idx_j)

    lane = lax.iota(jnp.int32, L)

    def chunk_body(ch, _):
        def fetch(g, _c):
            iv = idx_i[pl.ds(ch * CH + g * L, L)]
            jv = idx_j[pl.ds(ch * CH + g * L, L)]
            for t in range(L):
                pltpu.make_async_copy(
                    h_hbm.at[pl.ds(iv[t], 1), :],
                    hb.at[pl.ds(g * L + t, 1), :], sem).start()
                pltpu.make_async_copy(
                    c_hbm.at[pl.ds(jv[t], 1), :],
                    cb.at[pl.ds(g * L + t, 1), :], sem).start()
            return ()

        lax.fori_loop(0, GPC, fetch, (), unroll=False)

        def drain(r, _c):
            pltpu.make_async_copy(
                h_hbm.at[pl.ds(0, 1), :],
                hb.at[pl.ds(r, 1), :], sem).wait()
            pltpu.make_async_copy(
                c_hbm.at[pl.ds(0, 1), :],
                cb.at[pl.ds(r, 1), :], sem).wait()
            return ()

        lax.fori_loop(0, CH, drain, (), unroll=False)

        for lg in range(GPC):
            rows = lg * L + lane
            acc = jnp.zeros((L,), jnp.float32)
            dvec = jnp.zeros((L,), jnp.int32)
            for _step in range(D):
                hv = plsc.load_gather(hb, [rows, dvec])
                cv = plsc.load_gather(cb, [rows, dvec])
                acc = acc + hv * cv
                dvec = dvec + 1
            sig = 1.0 / (1.0 + jnp.exp(-acc))
            out_v[pl.ds(ch * CH + lg * L, L)] = sig
        return ()

    lax.fori_loop(0, NCHUNK, chunk_body, (), unroll=False)

    pltpu.sync_copy(out_v, out_hbm.at[pl.ds(base, B_PER_W)])


@jax.jit
def kernel(i, j, H, C):
    mesh = plsc.VectorSubcoreMesh(
        core_axis_name="c", subcore_axis_name="s",
        num_cores=NC, num_subcores=NS)
    run = pl.kernel(
        _body,
        out_type=jax.ShapeDtypeStruct((B,), jnp.float32),
        mesh=mesh,
        scratch_types=[
            pltpu.VMEM((B_PER_W,), jnp.int32),
            pltpu.VMEM((B_PER_W,), jnp.int32),
            pltpu.VMEM((CH, D), jnp.float32),
            pltpu.VMEM((CH, D), jnp.float32),
            pltpu.VMEM((B_PER_W,), jnp.float32),
            pltpu.SemaphoreType.DMA,
        ],
        compiler_params=pltpu.CompilerParams(needs_layout_passes=False),
    )
    return run(i.astype(jnp.int32), j.astype(jnp.int32), H, C)
